# baseline (speedup 1.0000x reference)
import functools

import jax
import jax.numpy as jnp
from jax import lax
from jax.experimental import pallas as pl
from jax.experimental.pallas import tpu as pltpu
from jax.experimental.pallas import tpu_sc as plsc

NC = 2
NS = 16
L = 16
NW = NC * NS

BATCH = 16384
EMB = 32
BPW = BATCH // NW


def _mf_body(user_hbm, item_hbm, ue_hbm, ie_hbm, ub_hbm, ib_hbm, out_hbm,
             idx_u, idx_i, urows, irows, ubias, ibias, outv,
             sem0, sem1, sem2, sem3):
    wid = lax.axis_index("s") * NC + lax.axis_index("c")
    base = wid * BPW

    pltpu.sync_copy(user_hbm.at[pl.ds(base, BPW)], idx_u)
    pltpu.sync_copy(item_hbm.at[pl.ds(base, BPW)], idx_i)

    c0 = pltpu.async_copy(ue_hbm.at[idx_u], urows, sem0)
    c1 = pltpu.async_copy(ie_hbm.at[idx_i], irows, sem1)
    c2 = pltpu.async_copy(ub_hbm.at[idx_u], ubias, sem2)
    c3 = pltpu.async_copy(ib_hbm.at[idx_i], ibias, sem3)
    c0.wait()
    c1.wait()
    c2.wait()
    c3.wait()

    lane = lax.iota(jnp.int32, L)
    zeros = jnp.zeros((L,), jnp.int32)

    def g_body(g, carry):
        rows = g * L + lane
        acc = plsc.load_gather(ubias, [rows])
        acc = acc + plsc.load_gather(ibias, [rows])
        for d in range(EMB):
            dcol = jnp.full((L,), d, jnp.int32)
            u = plsc.load_gather(urows, [rows, dcol])
            v = plsc.load_gather(irows, [rows, dcol])
            acc = acc + u * v
        outv[pl.ds(g * L, L)] = acc
        return carry

    lax.fori_loop(0, BPW // L, g_body, 0)

    pltpu.sync_copy(outv, out_hbm.at[pl.ds(base, BPW)])


@functools.partial(jax.jit, static_argnums=())
def _mf_call(user, item, user_emb_w, item_emb_w, user_bias_w, item_bias_w):
    mesh = plsc.VectorSubcoreMesh(core_axis_name="c", subcore_axis_name="s")
    run = pl.kernel(
        _mf_body,
        out_type=jax.ShapeDtypeStruct((BATCH,), jnp.float32),
        mesh=mesh,
        compiler_params=pltpu.CompilerParams(needs_layout_passes=False,
                                             use_tc_tiling_on_sc=False),
        scratch_types=[
            pltpu.VMEM((BPW,), jnp.int32),
            pltpu.VMEM((BPW,), jnp.int32),
            pltpu.VMEM((BPW, EMB), jnp.float32),
            pltpu.VMEM((BPW, EMB), jnp.float32),
            pltpu.VMEM((BPW,), jnp.float32),
            pltpu.VMEM((BPW,), jnp.float32),
            pltpu.VMEM((BPW,), jnp.float32),
            pltpu.SemaphoreType.DMA,
            pltpu.SemaphoreType.DMA,
            pltpu.SemaphoreType.DMA,
            pltpu.SemaphoreType.DMA,
        ],
    )
    return run(user, item, user_emb_w, item_emb_w, user_bias_w, item_bias_w)


def kernel(user, item, user_emb_w, item_emb_w, user_bias_w, item_bias_w):
    user = user.astype(jnp.int32)
    item = item.astype(jnp.int32)
    return _mf_call(user, item, user_emb_w, item_emb_w,
                    user_bias_w.reshape(-1), item_bias_w.reshape(-1))
